# initial kernel scaffold (unmeasured)
import jax
import jax.numpy as jnp
from jax import lax
from jax.experimental import pallas as pl
from jax.experimental.pallas import tpu as pltpu

N_DEV = 32


def kernel(A, B):
    m_per, k = A.shape
    _, n = B.shape
    m_total = N_DEV * m_per

    def body(a_ref, b_ref, out_ref, comm_ref, send_sems, recv_sems, copy_sem):
        my = lax.axis_index("i")
        left = (my + N_DEV - 1) % N_DEV
        right = (my + 1) % N_DEV

        barrier_sem = pltpu.get_barrier_semaphore()
        for nbr in (left, right):
            pl.semaphore_signal(
                barrier_sem, inc=1,
                device_id=(nbr,), device_id_type=pl.DeviceIdType.MESH,
            )
        pl.semaphore_wait(barrier_sem, 2)

        comm_ref[0, :, :] = jnp.dot(
            a_ref[:, :], b_ref[:, :], preferred_element_type=jnp.float32
        )
        cp = pltpu.make_async_copy(
            comm_ref.at[0], out_ref.at[pl.ds(my * m_per, m_per), :], copy_sem
        )
        cp.start()
        cp.wait()

        for h in range(N_DEV - 1):
            s = h % 2
            r = (h + 1) % 2
            rdma = pltpu.make_async_remote_copy(
                src_ref=comm_ref.at[s],
                dst_ref=comm_ref.at[r],
                send_sem=send_sems.at[s],
                recv_sem=recv_sems.at[r],
                device_id=(right,),
                device_id_type=pl.DeviceIdType.MESH,
            )
            rdma.start()
            rdma.wait()

            origin = (my - (h + 1) + N_DEV) % N_DEV
            cp = pltpu.make_async_copy(
                comm_ref.at[r],
                out_ref.at[pl.ds(origin * m_per, m_per), :],
                copy_sem,
            )
            cp.start()
            cp.wait()

    return pl.pallas_call(
        body,
        out_shape=jax.ShapeDtypeStruct((m_total, n), jnp.float32),
        in_specs=[
            pl.BlockSpec(memory_space=pltpu.VMEM),
            pl.BlockSpec(memory_space=pltpu.VMEM),
        ],
        out_specs=pl.BlockSpec(memory_space=pltpu.ANY),
        scratch_shapes=[
            pltpu.VMEM((2, m_per, n), jnp.float32),
            pltpu.SemaphoreType.DMA((2,)),
            pltpu.SemaphoreType.DMA((2,)),
            pltpu.SemaphoreType.DMA,
        ],
        compiler_params=pltpu.CompilerParams(collective_id=0),
    )(A, B)


# baseline (device time: 3515666 ns/iter reference)
import jax
import jax.numpy as jnp
from jax import lax
from jax.experimental import pallas as pl
from jax.experimental.pallas import tpu as pltpu

N_DEV = 32


def kernel(A, B):
    m_per, k = A.shape
    _, n = B.shape
    m_total = N_DEV * m_per

    def body(a_ref, b_ref, out_ref, comm_ref, send_sems, recv_sems, copy_sem):
        my = lax.axis_index("i")
        left = (my + N_DEV - 1) % N_DEV
        right = (my + 1) % N_DEV

        barrier_sem = pltpu.get_barrier_semaphore()
        for nbr in (left, right):
            pl.semaphore_signal(
                barrier_sem, inc=1,
                device_id=(nbr,), device_id_type=pl.DeviceIdType.MESH,
            )
        pl.semaphore_wait(barrier_sem, 2)

        comm_ref[0, :, :] = jnp.dot(
            a_ref[:, :], b_ref[:, :], preferred_element_type=jnp.float32
        )
        cp = pltpu.make_async_copy(
            comm_ref.at[0], out_ref.at[pl.ds(my * m_per, m_per), :], copy_sem
        )
        cp.start()
        cp.wait()

        for h in range(N_DEV - 1):
            s = h % 2
            r = (h + 1) % 2
            rdma = pltpu.make_async_remote_copy(
                src_ref=comm_ref.at[s],
                dst_ref=comm_ref.at[r],
                send_sem=send_sems.at[s],
                recv_sem=recv_sems.at[r],
                device_id=(right,),
                device_id_type=pl.DeviceIdType.MESH,
            )
            rdma.start()
            rdma.wait()

            origin = (my - (h + 1) + N_DEV) % N_DEV
            cp = pltpu.make_async_copy(
                comm_ref.at[r],
                out_ref.at[pl.ds(origin * m_per, m_per), :],
                copy_sem,
            )
            cp.start()
            cp.wait()

    return pl.pallas_call(
        body,
        out_shape=jax.ShapeDtypeStruct((m_total, n), jnp.float32),
        in_specs=[
            pl.BlockSpec(memory_space=pltpu.VMEM),
            pl.BlockSpec(memory_space=pltpu.VMEM),
        ],
        out_specs=pl.BlockSpec(memory_space=pl.ANY),
        scratch_shapes=[
            pltpu.VMEM((2, m_per, n), jnp.float32),
            pltpu.SemaphoreType.DMA((2,)),
            pltpu.SemaphoreType.DMA((2,)),
            pltpu.SemaphoreType.DMA,
        ],
        compiler_params=pltpu.CompilerParams(collective_id=0),
    )(A, B)


# device time: 1870556 ns/iter; 1.8795x vs baseline; 1.8795x over previous
import jax
import jax.numpy as jnp
from jax import lax
from jax.experimental import pallas as pl
from jax.experimental.pallas import tpu as pltpu

N_DEV = 32
N_R = 16
N_L = 15

_C16 = [
    (0, 0), (0, 1), (0, 2), (0, 3), (1, 3), (2, 3), (3, 3), (3, 2),
    (2, 2), (1, 2), (1, 1), (2, 1), (3, 1), (3, 0), (2, 0), (1, 0),
]
_CYCLE = [(0, y, z) for (y, z) in _C16] + [(1, y, z) for (y, z) in reversed(_C16)]


def _logical_id(x: int, y: int, z: int) -> int:
    return 8 * z + 2 * y + (x if y % 2 == 0 else 1 - x)


RING = [_logical_id(*c) for c in _CYCLE]
assert sorted(RING) == list(range(N_DEV))


def kernel(A, B):
    m_per, _ = A.shape
    _, n = B.shape
    m_total = N_DEV * m_per

    my = lax.axis_index("i")
    ring = jnp.array(RING, dtype=jnp.int32)
    j = jnp.argmax(ring == my)
    right = ring[(j + 1) % N_DEV]
    left = ring[(j - 1) % N_DEV]
    origins_r = ring[(j - 1 - jnp.arange(N_R)) % N_DEV]
    origins_l = ring[(j + 1 + jnp.arange(N_L)) % N_DEV]
    params = jnp.concatenate(
        [right[None], left[None], origins_r, origins_l]
    ).astype(jnp.int32)

    def body(params_ref, a_ref, b_ref, out_ref, chunk_ref, copy_sem,
             send_r, recv_r, send_l, recv_l):
        my_id = lax.axis_index("i")
        rgt = params_ref[0]
        lft = params_ref[1]

        chunk_ref[:, :] = jnp.dot(
            a_ref[:, :], b_ref[:, :], preferred_element_type=jnp.float32
        )
        cp = pltpu.make_async_copy(
            chunk_ref, out_ref.at[pl.ds(my_id * m_per, m_per), :], copy_sem
        )
        cp.start()
        cp.wait()

        barrier_sem = pltpu.get_barrier_semaphore()
        for nbr in (lft, rgt):
            pl.semaphore_signal(
                barrier_sem, inc=1,
                device_id=(nbr,), device_id_type=pl.DeviceIdType.MESH,
            )
        pl.semaphore_wait(barrier_sem, 2)

        def out_slice(origin):
            return out_ref.at[pl.ds(origin * m_per, m_per), :]

        for h in range(N_R):
            src_r = my_id if h == 0 else params_ref[2 + (h - 1)]
            rdma_r = pltpu.make_async_remote_copy(
                src_ref=out_slice(src_r),
                dst_ref=out_slice(src_r),
                send_sem=send_r.at[h % 2],
                recv_sem=recv_r.at[h % 2],
                device_id=(rgt,),
                device_id_type=pl.DeviceIdType.MESH,
            )
            rdma_r.start()
            if h < N_L:
                src_l = my_id if h == 0 else params_ref[2 + N_R + (h - 1)]
                rdma_l = pltpu.make_async_remote_copy(
                    src_ref=out_slice(src_l),
                    dst_ref=out_slice(src_l),
                    send_sem=send_l.at[h % 2],
                    recv_sem=recv_l.at[h % 2],
                    device_id=(lft,),
                    device_id_type=pl.DeviceIdType.MESH,
                )
                rdma_l.start()
                rdma_l.wait()
            rdma_r.wait()

    return pl.pallas_call(
        body,
        out_shape=jax.ShapeDtypeStruct((m_total, n), jnp.float32),
        in_specs=[
            pl.BlockSpec(memory_space=pltpu.SMEM),
            pl.BlockSpec(memory_space=pltpu.VMEM),
            pl.BlockSpec(memory_space=pltpu.VMEM),
        ],
        out_specs=pl.BlockSpec(memory_space=pl.ANY),
        scratch_shapes=[
            pltpu.VMEM((m_per, n), jnp.float32),
            pltpu.SemaphoreType.DMA,
            pltpu.SemaphoreType.DMA((2,)),
            pltpu.SemaphoreType.DMA((2,)),
            pltpu.SemaphoreType.DMA((2,)),
            pltpu.SemaphoreType.DMA((2,)),
        ],
        compiler_params=pltpu.CompilerParams(collective_id=0),
    )(params, A, B)


# device time: 1864363 ns/iter; 1.8857x vs baseline; 1.0033x over previous
import jax
import jax.numpy as jnp
from jax import lax
from jax.experimental import pallas as pl
from jax.experimental.pallas import tpu as pltpu

N_DEV = 32
N_R = 16
N_L = 15

_C16 = [
    (0, 0), (0, 1), (0, 2), (0, 3), (1, 3), (2, 3), (3, 3), (3, 2),
    (2, 2), (1, 2), (1, 1), (2, 1), (3, 1), (3, 0), (2, 0), (1, 0),
]
_CYCLE = [(0, y, z) for (y, z) in _C16] + [(1, y, z) for (y, z) in reversed(_C16)]


def _logical_id(x: int, y: int, z: int) -> int:
    return 8 * z + 2 * y + (x if y % 2 == 0 else 1 - x)


RING = [_logical_id(*c) for c in _CYCLE]
assert sorted(RING) == list(range(N_DEV))


def kernel(A, B):
    m_per, _ = A.shape
    _, n = B.shape
    m_total = N_DEV * m_per

    my = lax.axis_index("i")
    ring = jnp.array(RING, dtype=jnp.int32)
    j = jnp.argmax(ring == my)
    right = ring[(j + 1) % N_DEV]
    left = ring[(j - 1) % N_DEV]
    origins_r = ring[(j - 1 - jnp.arange(N_R)) % N_DEV]
    origins_l = ring[(j + 1 + jnp.arange(N_L)) % N_DEV]
    params = jnp.concatenate(
        [right[None], left[None], origins_r, origins_l]
    ).astype(jnp.int32)

    def body(params_ref, a_ref, b_ref, out_ref, chunk_ref, copy_sem,
             send_r, recv_r, send_l, recv_l):
        my_id = lax.axis_index("i")
        rgt = params_ref[0]
        lft = params_ref[1]

        chunk_ref[:, :] = jnp.dot(
            a_ref[:, :], b_ref[:, :], preferred_element_type=jnp.float32
        )
        cp = pltpu.make_async_copy(
            chunk_ref, out_ref.at[pl.ds(my_id * m_per, m_per), :], copy_sem
        )
        cp.start()

        barrier_sem = pltpu.get_barrier_semaphore()
        for nbr in (lft, rgt):
            pl.semaphore_signal(
                barrier_sem, inc=1,
                device_id=(nbr,), device_id_type=pl.DeviceIdType.MESH,
            )
        pl.semaphore_wait(barrier_sem, 2)

        def out_slice(origin):
            return out_ref.at[pl.ds(origin * m_per, m_per), :]

        def mk(h, n_hops, stream_base, ssem, rsem, tgt):
            src = (
                chunk_ref
                if h == 0
                else out_slice(params_ref[stream_base + (h - 1)])
            )
            dst = out_slice(my_id) if h == 0 else src
            return pltpu.make_async_remote_copy(
                src_ref=src,
                dst_ref=dst,
                send_sem=ssem.at[h % 2],
                recv_sem=rsem.at[h % 2],
                device_id=(tgt,),
                device_id_type=pl.DeviceIdType.MESH,
            )

        r_descs = [mk(h, N_R, 2, send_r, recv_r, rgt) for h in range(N_R)]
        l_descs = [mk(h, N_L, 2 + N_R, send_l, recv_l, lft) for h in range(N_L)]
        r_descs[0].start()
        l_descs[0].start()
        for h in range(1, N_R):
            r_descs[h - 1].wait_recv()
            if h >= 2:
                r_descs[h - 2].wait_send()
            r_descs[h].start()
            if h < N_L:
                l_descs[h - 1].wait_recv()
                if h >= 2:
                    l_descs[h - 2].wait_send()
                l_descs[h].start()
        r_descs[N_R - 1].wait_recv()
        l_descs[N_L - 1].wait_recv()
        for d in (r_descs[N_R - 2], r_descs[N_R - 1],
                  l_descs[N_L - 2], l_descs[N_L - 1]):
            d.wait_send()
        cp.wait()

    return pl.pallas_call(
        body,
        out_shape=jax.ShapeDtypeStruct((m_total, n), jnp.float32),
        in_specs=[
            pl.BlockSpec(memory_space=pltpu.SMEM),
            pl.BlockSpec(memory_space=pltpu.VMEM),
            pl.BlockSpec(memory_space=pltpu.VMEM),
        ],
        out_specs=pl.BlockSpec(memory_space=pl.ANY),
        scratch_shapes=[
            pltpu.VMEM((m_per, n), jnp.float32),
            pltpu.SemaphoreType.DMA,
            pltpu.SemaphoreType.DMA((2,)),
            pltpu.SemaphoreType.DMA((2,)),
            pltpu.SemaphoreType.DMA((2,)),
            pltpu.SemaphoreType.DMA((2,)),
        ],
        compiler_params=pltpu.CompilerParams(collective_id=0),
    )(params, A, B)


# device time: 1861389 ns/iter; 1.8887x vs baseline; 1.0016x over previous
import jax
import jax.numpy as jnp
from jax import lax
from jax.experimental import pallas as pl
from jax.experimental.pallas import tpu as pltpu

N_DEV = 32
N_R = 16
N_L = 15

_C16 = [
    (0, 0), (0, 1), (0, 2), (0, 3), (1, 3), (2, 3), (3, 3), (3, 2),
    (2, 2), (1, 2), (1, 1), (2, 1), (3, 1), (3, 0), (2, 0), (1, 0),
]
_CYCLE = [(0, y, z) for (y, z) in _C16] + [(1, y, z) for (y, z) in reversed(_C16)]


def _logical_id(x: int, y: int, z: int) -> int:
    return 8 * z + 2 * y + (x if y % 2 == 0 else 1 - x)


RING = [_logical_id(*c) for c in _CYCLE]
assert sorted(RING) == list(range(N_DEV))


def kernel(A, B):
    m_per, _ = A.shape
    _, n = B.shape
    m_total = N_DEV * m_per

    my = lax.axis_index("i")
    ring = jnp.array(RING, dtype=jnp.int32)
    j = jnp.argmax(ring == my)
    right = ring[(j + 1) % N_DEV]
    left = ring[(j - 1) % N_DEV]
    origins_r = ring[(j - 1 - jnp.arange(N_R)) % N_DEV]
    origins_l = ring[(j + 1 + jnp.arange(N_L)) % N_DEV]
    params = jnp.concatenate(
        [right[None], left[None], origins_r, origins_l]
    ).astype(jnp.int32)

    def body(params_ref, a_ref, b_ref, out_ref, chunk_ref, comm_r, comm_l,
             copy_sem, copy_sems_r, copy_sems_l,
             send_r, recv_r, send_l, recv_l):
        my_id = lax.axis_index("i")
        rgt = params_ref[0]
        lft = params_ref[1]

        chunk_ref[:, :] = jnp.dot(
            a_ref[:, :], b_ref[:, :], preferred_element_type=jnp.float32
        )
        cp = pltpu.make_async_copy(
            chunk_ref, out_ref.at[pl.ds(my_id * m_per, m_per), :], copy_sem
        )
        cp.start()

        barrier_sem = pltpu.get_barrier_semaphore()
        for nbr in (lft, rgt):
            pl.semaphore_signal(
                barrier_sem, inc=1,
                device_id=(nbr,), device_id_type=pl.DeviceIdType.MESH,
            )
        pl.semaphore_wait(barrier_sem, 2)

        def out_slice(origin):
            return out_ref.at[pl.ds(origin * m_per, m_per), :]

        def mk(h, comm, ssem, rsem, tgt):
            src = chunk_ref if h == 0 else comm.at[(h - 1) % 2]
            return pltpu.make_async_remote_copy(
                src_ref=src,
                dst_ref=comm.at[h % 2],
                send_sem=ssem.at[h % 2],
                recv_sem=rsem.at[h % 2],
                device_id=(tgt,),
                device_id_type=pl.DeviceIdType.MESH,
            )

        r_descs = [mk(h, comm_r, send_r, recv_r, rgt) for h in range(N_R)]
        l_descs = [mk(h, comm_l, send_l, recv_l, lft) for h in range(N_L)]
        copies_r, copies_l = [], []

        def advance(h, descs, n_hops, comm, csems, copies, stream_base):
            descs[h - 1].wait_recv()
            if h >= 2:
                descs[h - 2].wait_send()
            if h < n_hops:
                descs[h].start()
            if h >= 3:
                copies[h - 3].wait()
            cpd = pltpu.make_async_copy(
                comm.at[(h - 1) % 2],
                out_slice(params_ref[stream_base + (h - 1)]),
                csems.at[(h - 1) % 2],
            )
            cpd.start()
            copies.append(cpd)

        r_descs[0].start()
        l_descs[0].start()
        for h in range(1, N_R + 1):
            advance(h, r_descs, N_R, comm_r, copy_sems_r, copies_r, 2)
            if h <= N_L:
                advance(h, l_descs, N_L, comm_l, copy_sems_l, copies_l, 2 + N_R)
        r_descs[N_R - 1].wait_send()
        l_descs[N_L - 1].wait_send()
        for cpd in copies_r[-2:] + copies_l[-2:]:
            cpd.wait()
        cp.wait()

    return pl.pallas_call(
        body,
        out_shape=jax.ShapeDtypeStruct((m_total, n), jnp.float32),
        in_specs=[
            pl.BlockSpec(memory_space=pltpu.SMEM),
            pl.BlockSpec(memory_space=pltpu.VMEM),
            pl.BlockSpec(memory_space=pltpu.VMEM),
        ],
        out_specs=pl.BlockSpec(memory_space=pl.ANY),
        scratch_shapes=[
            pltpu.VMEM((m_per, n), jnp.float32),
            pltpu.VMEM((2, m_per, n), jnp.float32),
            pltpu.VMEM((2, m_per, n), jnp.float32),
            pltpu.SemaphoreType.DMA,
            pltpu.SemaphoreType.DMA((2,)),
            pltpu.SemaphoreType.DMA((2,)),
            pltpu.SemaphoreType.DMA((2,)),
            pltpu.SemaphoreType.DMA((2,)),
            pltpu.SemaphoreType.DMA((2,)),
            pltpu.SemaphoreType.DMA((2,)),
        ],
        compiler_params=pltpu.CompilerParams(
            collective_id=0, vmem_limit_bytes=100 * 1024 * 1024
        ),
    )(params, A, B)


# device time: 1480549 ns/iter; 2.3746x vs baseline; 1.2572x over previous
import jax
import jax.numpy as jnp
from jax import lax
from jax.experimental import pallas as pl
from jax.experimental.pallas import tpu as pltpu

N_DEV = 32
N_PAIR = 16
N_R = 8
N_L = 7
SW = 1024
XW = 512

_C16 = [
    (0, 0), (0, 1), (0, 2), (0, 3), (1, 3), (2, 3), (3, 3), (3, 2),
    (2, 2), (1, 2), (1, 1), (2, 1), (3, 1), (3, 0), (2, 0), (1, 0),
]
C16Q = [4 * z + y for (y, z) in _C16]
assert sorted(C16Q) == list(range(N_PAIR))


def kernel(A, B):
    m_per, _ = A.shape
    _, n = B.shape
    m_total = N_DEV * m_per
    pair_m = 2 * m_per

    my = lax.axis_index("i")
    q_me = my // 2
    y = q_me % 4
    xbit = jnp.where(y % 2 == 0, my % 2, 1 - my % 2)
    partner = my + 1 - 2 * (my % 2)

    c16q = jnp.array(C16Q, dtype=jnp.int32)
    jc = jnp.argmax(c16q == q_me)

    def lid_from_q(q, xb):
        yy = q % 4
        bit = jnp.where(yy % 2 == 0, xb, 1 - xb)
        return 2 * q + bit

    cright = lid_from_q(c16q[(jc + 1) % N_PAIR], xbit)
    cleft = lid_from_q(c16q[(jc - 1) % N_PAIR], xbit)
    c0 = xbit * (n - SW)
    c0x = xbit * (n - XW)
    p_r = c16q[(jc - 1 - jnp.arange(N_R)) % N_PAIR]
    p_l = c16q[(jc + 1 + jnp.arange(N_L)) % N_PAIR]
    params = jnp.concatenate(
        [
            jnp.stack([partner, cright, cleft, c0, c0x]),
            p_r,
            p_l,
        ]
    ).astype(jnp.int32)

    def body(params_ref, a_ref, b_ref, out_ref, chunk_ref,
             copy_sem, ex_s, ex_r, s_r, r_r, s_l, r_l, s_x, r_x):
        my_id = lax.axis_index("i")
        q_b = my_id // 2
        prt = params_ref[0]
        crt = params_ref[1]
        clt = params_ref[2]
        col0 = pl.multiple_of(params_ref[3], 512)
        colx = pl.multiple_of(params_ref[4], 512)

        chunk_ref[:, :] = jnp.dot(
            a_ref[:, :], b_ref[:, :], preferred_element_type=jnp.float32
        )
        cp = pltpu.make_async_copy(
            chunk_ref, out_ref.at[pl.ds(my_id * m_per, m_per), :], copy_sem
        )
        cp.start()

        barrier_sem = pltpu.get_barrier_semaphore()
        for nbr in (prt, clt, crt):
            pl.semaphore_signal(
                barrier_sem, inc=1,
                device_id=(nbr,), device_id_type=pl.DeviceIdType.MESH,
            )
        pl.semaphore_wait(barrier_sem, 3)

        ex = pltpu.make_async_remote_copy(
            src_ref=chunk_ref,
            dst_ref=out_ref.at[pl.ds(my_id * m_per, m_per), :],
            send_sem=ex_s,
            recv_sem=ex_r,
            device_id=(prt,),
            device_id_type=pl.DeviceIdType.MESH,
        )
        ex.start()

        def prows(p, cstart, w):
            return out_ref.at[pl.ds(p * pair_m, pair_m), pl.ds(cstart, w)]

        def mk_cyc(h, origins_base, ssem, rsem, tgt):
            src_pair = q_b if h == 0 else params_ref[origins_base + (h - 1)]
            sl = prows(src_pair, col0, SW)
            return pltpu.make_async_remote_copy(
                src_ref=sl, dst_ref=sl,
                send_sem=ssem.at[h % 2], recv_sem=rsem.at[h % 2],
                device_id=(tgt,), device_id_type=pl.DeviceIdType.MESH,
            )

        cyc_r = [mk_cyc(h, 5, s_r, r_r, crt) for h in range(N_R)]
        cyc_l = [mk_cyc(h, 13, s_l, r_l, clt) for h in range(N_L)]

        x_descs = []

        def xfwd(p):
            k = len(x_descs)
            sl = prows(p, colx, XW)
            d = pltpu.make_async_remote_copy(
                src_ref=sl, dst_ref=sl,
                send_sem=s_x.at[k % 2], recv_sem=r_x.at[k % 2],
                device_id=(prt,), device_id_type=pl.DeviceIdType.MESH,
            )
            if k >= 2:
                x_descs[k - 2].wait_send()
            d.start()
            x_descs.append(d)

        ex.wait_recv()
        cp.wait()
        cyc_r[0].start()
        cyc_l[0].start()
        for t in range(1, N_R + 1):
            cyc_r[t - 1].wait_recv()
            if t >= 2:
                cyc_r[t - 2].wait_send()
            if t < N_R:
                cyc_r[t].start()
            xfwd(params_ref[5 + (t - 1)])
            if t <= N_L:
                cyc_l[t - 1].wait_recv()
                if t >= 2:
                    cyc_l[t - 2].wait_send()
                if t < N_L:
                    cyc_l[t].start()
                xfwd(params_ref[13 + (t - 1)])

        cyc_r[N_R - 1].wait_send()
        cyc_l[N_L - 1].wait_send()
        ex.wait_send()
        x_descs[-2].wait_send()
        x_descs[-1].wait_send()
        for d in x_descs:
            d.wait_recv()

    return pl.pallas_call(
        body,
        out_shape=jax.ShapeDtypeStruct((m_total, n), jnp.float32),
        in_specs=[
            pl.BlockSpec(memory_space=pltpu.SMEM),
            pl.BlockSpec(memory_space=pltpu.VMEM),
            pl.BlockSpec(memory_space=pltpu.VMEM),
        ],
        out_specs=pl.BlockSpec(memory_space=pl.ANY),
        scratch_shapes=[
            pltpu.VMEM((m_per, n), jnp.float32),
            pltpu.SemaphoreType.DMA,
            pltpu.SemaphoreType.DMA,
            pltpu.SemaphoreType.DMA,
            pltpu.SemaphoreType.DMA((2,)),
            pltpu.SemaphoreType.DMA((2,)),
            pltpu.SemaphoreType.DMA((2,)),
            pltpu.SemaphoreType.DMA((2,)),
            pltpu.SemaphoreType.DMA((2,)),
            pltpu.SemaphoreType.DMA((2,)),
        ],
        compiler_params=pltpu.CompilerParams(
            collective_id=0, vmem_limit_bytes=100 * 1024 * 1024
        ),
    )(params, A, B)


# device time: 1413280 ns/iter; 2.4876x vs baseline; 1.0476x over previous
import jax
import jax.numpy as jnp
from jax import lax
from jax.experimental import pallas as pl
from jax.experimental.pallas import tpu as pltpu

N_DEV = 32
N_PAIR = 16
N_R = 8
N_L = 7
SW = 1024
XW = 512

_C16 = [
    (0, 0), (0, 1), (0, 2), (0, 3), (1, 3), (2, 3), (3, 3), (3, 2),
    (2, 2), (1, 2), (1, 1), (2, 1), (3, 1), (3, 0), (2, 0), (1, 0),
]
C16Q = [4 * z + y for (y, z) in _C16]
assert sorted(C16Q) == list(range(N_PAIR))


def kernel(A, B):
    m_per, _ = A.shape
    _, n = B.shape
    m_total = N_DEV * m_per
    pair_m = 2 * m_per

    my = lax.axis_index("i")
    q_me = my // 2
    y = q_me % 4
    xbit = jnp.where(y % 2 == 0, my % 2, 1 - my % 2)
    partner = my + 1 - 2 * (my % 2)

    c16q = jnp.array(C16Q, dtype=jnp.int32)
    jc = jnp.argmax(c16q == q_me)

    def lid_from_q(q, xb):
        yy = q % 4
        bit = jnp.where(yy % 2 == 0, xb, 1 - xb)
        return 2 * q + bit

    cright = lid_from_q(c16q[(jc + 1) % N_PAIR], xbit)
    cleft = lid_from_q(c16q[(jc - 1) % N_PAIR], xbit)
    c0 = xbit * (n - SW)
    c0x = xbit * (n - XW)
    p_r = c16q[(jc - 1 - jnp.arange(N_R)) % N_PAIR]
    p_l = c16q[(jc + 1 + jnp.arange(N_L)) % N_PAIR]
    params = jnp.concatenate(
        [
            jnp.stack([partner, cright, cleft, c0, c0x]),
            p_r,
            p_l,
        ]
    ).astype(jnp.int32)

    def body(params_ref, a_ref, b_ref, out_ref, chunk_ref,
             copy_sem, ex_s, ex_r, s_r, r_r, s_l, r_l, s_x, r_x,
             s0b_r, r0b_r, s0b_l, r0b_l):
        my_id = lax.axis_index("i")
        q_b = my_id // 2
        prt = params_ref[0]
        crt = params_ref[1]
        clt = params_ref[2]
        col0 = pl.multiple_of(params_ref[3], 512)
        colx = pl.multiple_of(params_ref[4], 512)

        chunk_ref[:, :] = jnp.dot(
            a_ref[:, :], b_ref[:, :], preferred_element_type=jnp.float32
        )
        cp = pltpu.make_async_copy(
            chunk_ref, out_ref.at[pl.ds(my_id * m_per, m_per), :], copy_sem
        )
        cp.start()

        barrier_sem = pltpu.get_barrier_semaphore()
        for nbr in (prt, clt, crt):
            pl.semaphore_signal(
                barrier_sem, inc=1,
                device_id=(nbr,), device_id_type=pl.DeviceIdType.MESH,
            )
        pl.semaphore_wait(barrier_sem, 3)

        ex = pltpu.make_async_remote_copy(
            src_ref=chunk_ref,
            dst_ref=out_ref.at[pl.ds(my_id * m_per, m_per), :],
            send_sem=ex_s,
            recv_sem=ex_r,
            device_id=(prt,),
            device_id_type=pl.DeviceIdType.MESH,
        )
        ex.start()

        def prows(p, cstart, w):
            return out_ref.at[pl.ds(p * pair_m, pair_m), pl.ds(cstart, w)]

        def mk_cyc(h, origins_base, ssem, rsem, tgt):
            sl = prows(params_ref[origins_base + (h - 1)], col0, SW)
            return pltpu.make_async_remote_copy(
                src_ref=sl, dst_ref=sl,
                send_sem=ssem.at[h % 2], recv_sem=rsem.at[h % 2],
                device_id=(tgt,), device_id_type=pl.DeviceIdType.MESH,
            )

        def mk_h0(rows_lid, src, ssem, rsem, tgt):
            dst = out_ref.at[pl.ds(rows_lid * m_per, m_per), pl.ds(col0, SW)]
            return pltpu.make_async_remote_copy(
                src_ref=src, dst_ref=dst,
                send_sem=ssem, recv_sem=rsem,
                device_id=(tgt,), device_id_type=pl.DeviceIdType.MESH,
            )

        chunk_stripe = chunk_ref.at[:, pl.ds(col0, SW)]
        prt_stripe = out_ref.at[pl.ds(prt * m_per, m_per), pl.ds(col0, SW)]
        d0a_r = mk_h0(my_id, chunk_stripe, s_r.at[0], r_r.at[0], crt)
        d0b_r = mk_h0(prt, prt_stripe, s0b_r, r0b_r, crt)
        d0a_l = mk_h0(my_id, chunk_stripe, s_l.at[0], r_l.at[0], clt)
        d0b_l = mk_h0(prt, prt_stripe, s0b_l, r0b_l, clt)
        cyc_r = [None] + [mk_cyc(h, 5, s_r, r_r, crt) for h in range(1, N_R)]
        cyc_l = [None] + [mk_cyc(h, 13, s_l, r_l, clt) for h in range(1, N_L)]

        x_descs = []

        def xfwd(p):
            k = len(x_descs)
            sl = prows(p, colx, XW)
            d = pltpu.make_async_remote_copy(
                src_ref=sl, dst_ref=sl,
                send_sem=s_x.at[k % 2], recv_sem=r_x.at[k % 2],
                device_id=(prt,), device_id_type=pl.DeviceIdType.MESH,
            )
            if k >= 2:
                x_descs[k - 2].wait_send()
            d.start()
            x_descs.append(d)

        d0a_r.start()
        d0a_l.start()
        ex.wait_recv()
        d0b_r.start()
        d0b_l.start()
        for t in range(1, N_R + 1):
            if t == 1:
                d0a_r.wait_recv()
                d0b_r.wait_recv()
            else:
                cyc_r[t - 1].wait_recv()
            if t == 2:
                d0a_r.wait_send()
            elif t >= 3:
                cyc_r[t - 2].wait_send()
            if t < N_R:
                cyc_r[t].start()
            xfwd(params_ref[5 + (t - 1)])
            if t <= N_L:
                if t == 1:
                    d0a_l.wait_recv()
                    d0b_l.wait_recv()
                else:
                    cyc_l[t - 1].wait_recv()
                if t == 2:
                    d0a_l.wait_send()
                elif t >= 3:
                    cyc_l[t - 2].wait_send()
                if t < N_L:
                    cyc_l[t].start()
                xfwd(params_ref[13 + (t - 1)])

        cyc_r[N_R - 1].wait_send()
        cyc_l[N_L - 1].wait_send()
        d0b_r.wait_send()
        d0b_l.wait_send()
        ex.wait_send()
        x_descs[-2].wait_send()
        x_descs[-1].wait_send()
        for d in x_descs:
            d.wait_recv()
        cp.wait()

    return pl.pallas_call(
        body,
        out_shape=jax.ShapeDtypeStruct((m_total, n), jnp.float32),
        in_specs=[
            pl.BlockSpec(memory_space=pltpu.SMEM),
            pl.BlockSpec(memory_space=pltpu.VMEM),
            pl.BlockSpec(memory_space=pltpu.VMEM),
        ],
        out_specs=pl.BlockSpec(memory_space=pl.ANY),
        scratch_shapes=[
            pltpu.VMEM((m_per, n), jnp.float32),
            pltpu.SemaphoreType.DMA,
            pltpu.SemaphoreType.DMA,
            pltpu.SemaphoreType.DMA,
            pltpu.SemaphoreType.DMA((2,)),
            pltpu.SemaphoreType.DMA((2,)),
            pltpu.SemaphoreType.DMA((2,)),
            pltpu.SemaphoreType.DMA((2,)),
            pltpu.SemaphoreType.DMA((2,)),
            pltpu.SemaphoreType.DMA((2,)),
            pltpu.SemaphoreType.DMA,
            pltpu.SemaphoreType.DMA,
            pltpu.SemaphoreType.DMA,
            pltpu.SemaphoreType.DMA,
        ],
        compiler_params=pltpu.CompilerParams(
            collective_id=0, vmem_limit_bytes=100 * 1024 * 1024
        ),
    )(params, A, B)
